# Initial kernel scaffold; baseline (speedup 1.0000x reference)
import jax
import jax.numpy as jnp
from jax.experimental import pallas as pl


def kernel(node_deg, edge_index, embed_table, W1, b1, eps1, W2, b2, eps2, W3, b3, eps3, g1, be1, g2, be2):
    raise NotImplementedError("write your pallas kernel here")


# speedup vs baseline: 7.4919x; 7.4919x over previous
import functools

import jax
import jax.numpy as jnp
from jax import lax
from jax.experimental import pallas as pl
from jax.experimental.pallas import tpu as pltpu
from jax.experimental.pallas import tpu_sc as plsc

N = 10000
E = 320000
D = 128
CP = 80

NC = 2
NS = 16
EPT = E // (NC * NS)
CHUNK = 80
NCHUNK = EPT // CHUNK
NP = 10240
RPT = NP // NS
CFL = N * CP
CPT = CFL // NS

_mesh = plsc.VectorSubcoreMesh(core_axis_name="c", subcore_axis_name="s")


def _hist_sc(node_deg, src, dst):
    @functools.partial(
        pl.kernel,
        out_type=jax.ShapeDtypeStruct((NC * CFL,), jnp.float32),
        mesh=_mesh,
        compiler_params=pltpu.CompilerParams(needs_layout_passes=False),
        scratch_types=[
            pltpu.VMEM((N,), jnp.int32),
            pltpu.VMEM((EPT,), jnp.int32),
            pltpu.VMEM((EPT,), jnp.int32),
            pltpu.VMEM((CHUNK,), jnp.int32),
            pltpu.VMEM((CHUNK,), jnp.float32),
            pltpu.VMEM((EPT,), jnp.float32),
            pltpu.VMEM_SHARED((CFL,), jnp.float32),
        ],
    )
    def k(nd_hbm, src_hbm, dst_hbm, out_hbm, nd_v, src_v, dst_v,
          idx_v, ones_v, zv, acc):
        cid = lax.axis_index("c")
        sid = lax.axis_index("s")
        ebase = (cid * NS + sid) * EPT
        pltpu.sync_copy(nd_hbm, nd_v)
        pltpu.sync_copy(src_hbm.at[pl.ds(ebase, EPT)], src_v)
        pltpu.sync_copy(dst_hbm.at[pl.ds(ebase, EPT)], dst_v)

        def zbody(i, _):
            zv[pl.ds(i * 16, 16)] = jnp.zeros((16,), jnp.float32)
            return ()

        lax.fori_loop(0, EPT // 16, zbody, (), unroll=False)
        for q in range(CPT // EPT):
            pltpu.sync_copy(zv, acc.at[pl.ds(sid * CPT + q * EPT, EPT)])
        for j in range(CHUNK // 16):
            ones_v[pl.ds(j * 16, 16)] = jnp.ones((16,), jnp.float32)
        plsc.subcore_barrier()

        def body(c, _):
            for j in range(CHUNK // 16):
                o = c * CHUNK + j * 16
                s16 = src_v[pl.ds(o, 16)]
                d16 = dst_v[pl.ds(o, 16)]
                deg16 = plsc.load_gather(nd_v, [s16])
                idx_v[pl.ds(j * 16, 16)] = d16 * CP + deg16
            pltpu.sync_copy(ones_v, acc.at[idx_v], add=True)
            return ()

        lax.fori_loop(0, NCHUNK, body, (), unroll=False)
        plsc.subcore_barrier()
        for q in range(CPT // EPT):
            pltpu.sync_copy(acc.at[pl.ds(sid * CPT + q * EPT, EPT)], zv)
            pltpu.sync_copy(zv, out_hbm.at[
                pl.ds(cid * CFL + sid * CPT + q * EPT, EPT)])

    return k(node_deg, src, dst)


def _segsum_sc(x, src, dst):
    @functools.partial(
        pl.kernel,
        out_type=jax.ShapeDtypeStruct((NC, NP, D), jnp.float32),
        mesh=_mesh,
        compiler_params=pltpu.CompilerParams(needs_layout_passes=False),
        scratch_types=[
            pltpu.VMEM((EPT,), jnp.int32),
            pltpu.VMEM((EPT,), jnp.int32),
            pltpu.VMEM((CHUNK,), jnp.int32),
            pltpu.VMEM((CHUNK,), jnp.int32),
            pltpu.VMEM((CHUNK, D), jnp.float32),
            pltpu.VMEM((CHUNK, D), jnp.float32),
            pltpu.VMEM_SHARED((NP, D), jnp.float32),
            pltpu.SemaphoreType.DMA,
        ],
    )
    def k(x_hbm, src_hbm, dst_hbm, out_hbm, src_v, dst_v, gi_v, si_v,
          rows_v, zv, acc, sem):
        cid = lax.axis_index("c")
        sid = lax.axis_index("s")
        ebase = (cid * NS + sid) * EPT
        rbase = sid * RPT
        pltpu.sync_copy(src_hbm.at[pl.ds(ebase, EPT)], src_v)
        pltpu.sync_copy(dst_hbm.at[pl.ds(ebase, EPT)], dst_v)

        def zbody(i, _):
            for j in range(D // 16):
                zv[i, pl.ds(j * 16, 16)] = jnp.zeros((16,), jnp.float32)
            return ()

        lax.fori_loop(0, CHUNK, zbody, (), unroll=False)
        for q in range(RPT // CHUNK):
            pltpu.sync_copy(zv, acc.at[pl.ds(rbase + q * CHUNK, CHUNK)])
        plsc.subcore_barrier()

        def body(c, _):
            for j in range(CHUNK // 16):
                o = c * CHUNK + j * 16
                gi_v[pl.ds(j * 16, 16)] = src_v[pl.ds(o, 16)]
                si_v[pl.ds(j * 16, 16)] = dst_v[pl.ds(o, 16)]
            pltpu.async_copy(x_hbm.at[gi_v], rows_v, sem).wait()
            pltpu.sync_copy(rows_v, acc.at[si_v], add=True)
            return ()

        lax.fori_loop(0, NCHUNK, body, (), unroll=False)
        plsc.subcore_barrier()
        for q in range(RPT // CHUNK):
            pltpu.sync_copy(acc.at[pl.ds(rbase + q * CHUNK, CHUNK)], zv)
            pltpu.sync_copy(zv, out_hbm.at[cid, pl.ds(rbase + q * CHUNK,
                                                      CHUNK)])

    return k(x, src, dst)


RB = 400
NRB = N // RB


def _l1_tc(degf, p0, p1, embed_pad, W1, b1, s1):
    def body(degf_r, p0_r, p1_r, emb_r, w_r, b_r, s_r, y_r, ssum_r, ssq_r,
             t_r):
        i = pl.program_id(0)

        @pl.when(i == 0)
        def _():
            t_r[...] = jnp.dot(emb_r[...], w_r[...].T,
                               preferred_element_type=jnp.float32)
            ssum_r[...] = jnp.zeros((1, D), jnp.float32)
            ssq_r[...] = jnp.zeros((1, D), jnp.float32)

        io = lax.broadcasted_iota(jnp.int32, (RB, CP), 1)
        onehot = jnp.where(degf_r[...].astype(jnp.int32) == io, s_r[0, 0], 0.0)
        ceff = p0_r[...] + p1_r[...] + onehot
        y = jnp.dot(ceff, t_r[...], preferred_element_type=jnp.float32)
        y = y + b_r[...]
        y_r[...] = y
        ssum_r[...] += jnp.sum(y, axis=0, keepdims=True)
        ssq_r[...] += jnp.sum(y * y, axis=0, keepdims=True)

    return pl.pallas_call(
        body,
        grid=(NRB,),
        in_specs=[
            pl.BlockSpec((RB, 1), lambda i: (i, 0)),
            pl.BlockSpec((RB, CP), lambda i: (i, 0)),
            pl.BlockSpec((RB, CP), lambda i: (i, 0)),
            pl.BlockSpec((CP, D), lambda i: (0, 0)),
            pl.BlockSpec((D, D), lambda i: (0, 0)),
            pl.BlockSpec((1, D), lambda i: (0, 0)),
            pl.BlockSpec(memory_space=pltpu.SMEM),
        ],
        out_specs=[
            pl.BlockSpec((RB, D), lambda i: (i, 0)),
            pl.BlockSpec((1, D), lambda i: (0, 0)),
            pl.BlockSpec((1, D), lambda i: (0, 0)),
        ],
        out_shape=[
            jax.ShapeDtypeStruct((N, D), jnp.float32),
            jax.ShapeDtypeStruct((1, D), jnp.float32),
            jax.ShapeDtypeStruct((1, D), jnp.float32),
        ],
        scratch_shapes=[pltpu.VMEM((CP, D), jnp.float32)],
    )(degf, p0, p1, embed_pad, W1, b1, s1)


def _conv_tc(x, p0, p1, W, b, s, want_stats):
    def body(x_r, p0_r, p1_r, w_r, b_r, s_r, y_r, ssum_r, ssq_r):
        i = pl.program_id(0)

        @pl.when(i == 0)
        def _():
            ssum_r[...] = jnp.zeros((1, D), jnp.float32)
            ssq_r[...] = jnp.zeros((1, D), jnp.float32)

        h = s_r[0, 0] * x_r[...] + p0_r[...] + p1_r[...]
        y = jnp.dot(h, w_r[...].T, preferred_element_type=jnp.float32)
        y = y + b_r[...]
        y_r[...] = y
        if want_stats:
            ssum_r[...] += jnp.sum(y, axis=0, keepdims=True)
            ssq_r[...] += jnp.sum(y * y, axis=0, keepdims=True)

    return pl.pallas_call(
        body,
        grid=(NRB,),
        in_specs=[
            pl.BlockSpec((RB, D), lambda i: (i, 0)),
            pl.BlockSpec((RB, D), lambda i: (i, 0)),
            pl.BlockSpec((RB, D), lambda i: (i, 0)),
            pl.BlockSpec((D, D), lambda i: (0, 0)),
            pl.BlockSpec((1, D), lambda i: (0, 0)),
            pl.BlockSpec(memory_space=pltpu.SMEM),
        ],
        out_specs=[
            pl.BlockSpec((RB, D), lambda i: (i, 0)),
            pl.BlockSpec((1, D), lambda i: (0, 0)),
            pl.BlockSpec((1, D), lambda i: (0, 0)),
        ],
        out_shape=[
            jax.ShapeDtypeStruct((N, D), jnp.float32),
            jax.ShapeDtypeStruct((1, D), jnp.float32),
            jax.ShapeDtypeStruct((1, D), jnp.float32),
        ],
    )(x, p0, p1, W, b, s)


def _bn_act_tc(y, ssum, ssq, g, be):
    def body(y_r, ssum_r, ssq_r, g_r, be_r, o_r):
        m = ssum_r[...] / N
        v = ssq_r[...] / N - m * m
        inv = g_r[...] * lax.rsqrt(v + 1e-5)
        xn = (y_r[...] - m) * inv + be_r[...]
        o_r[...] = jnp.where(xn >= 0, xn, 0.01 * xn)

    return pl.pallas_call(
        body,
        grid=(NRB,),
        in_specs=[
            pl.BlockSpec((RB, D), lambda i: (i, 0)),
            pl.BlockSpec((1, D), lambda i: (0, 0)),
            pl.BlockSpec((1, D), lambda i: (0, 0)),
            pl.BlockSpec((1, D), lambda i: (0, 0)),
            pl.BlockSpec((1, D), lambda i: (0, 0)),
        ],
        out_specs=pl.BlockSpec((RB, D), lambda i: (i, 0)),
        out_shape=jax.ShapeDtypeStruct((N, D), jnp.float32),
    )(y, ssum, ssq, g, be)


def kernel(node_deg, edge_index, embed_table, W1, b1, eps1, W2, b2, eps2,
           W3, b3, eps3, g1, be1, g2, be2):
    node_deg = node_deg.astype(jnp.int32)
    src = edge_index[0].astype(jnp.int32)
    dst = edge_index[1].astype(jnp.int32)
    embed_pad = jnp.zeros((CP, D), jnp.float32).at[:embed_table.shape[0]].set(
        embed_table)
    degf = node_deg.astype(jnp.float32).reshape(N, 1)
    b1r, b2r, b3r = b1.reshape(1, D), b2.reshape(1, D), b3.reshape(1, D)
    g1r, g2r = g1.reshape(1, D), g2.reshape(1, D)
    be1r, be2r = be1.reshape(1, D), be2.reshape(1, D)
    s1 = (1.0 + eps1).astype(jnp.float32).reshape(1, 1)
    s2 = (1.0 + eps2).astype(jnp.float32).reshape(1, 1)
    s3 = (1.0 + eps3).astype(jnp.float32).reshape(1, 1)

    cpart = _hist_sc(node_deg, src, dst).reshape(NC, N, CP)
    y1, ssum1, ssq1 = _l1_tc(degf, cpart[0], cpart[1], embed_pad, W1, b1r, s1)
    x1 = _bn_act_tc(y1, ssum1, ssq1, g1r, be1r)

    p = _segsum_sc(x1, src, dst)
    y2, ssum2, ssq2 = _conv_tc(x1, p[0], p[1], W2, b2r, s2, True)
    x2 = _bn_act_tc(y2, ssum2, ssq2, g2r, be2r)

    p = _segsum_sc(x2, src, dst)
    y3, _, _ = _conv_tc(x2, p[0], p[1], W3, b3r, s3, False)
    return y3


# speedup vs baseline: 11.5028x; 1.5354x over previous
import functools

import jax
import jax.numpy as jnp
from jax import lax
from jax.experimental import pallas as pl
from jax.experimental.pallas import tpu as pltpu
from jax.experimental.pallas import tpu_sc as plsc

N = 10000
E = 320000
D = 128
CP = 80

NC = 2
NS = 16
EPT = E // (NC * NS)
CHUNK = 80
NCHUNK = EPT // CHUNK
NP = 10240
RPT = NP // NS
CFL = N * CP
CPT = CFL // NS

_mesh = plsc.VectorSubcoreMesh(core_axis_name="c", subcore_axis_name="s")


def _hist_sc(node_deg, src, dst):
    @functools.partial(
        pl.kernel,
        out_type=jax.ShapeDtypeStruct((NC * CFL,), jnp.float32),
        mesh=_mesh,
        compiler_params=pltpu.CompilerParams(needs_layout_passes=False),
        scratch_types=[
            pltpu.VMEM((N,), jnp.int32),
            pltpu.VMEM((EPT,), jnp.int32),
            pltpu.VMEM((EPT,), jnp.int32),
            pltpu.VMEM((CHUNK,), jnp.int32),
            pltpu.VMEM((CHUNK,), jnp.float32),
            pltpu.VMEM((EPT,), jnp.float32),
            pltpu.VMEM_SHARED((CFL,), jnp.float32),
        ],
    )
    def k(nd_hbm, src_hbm, dst_hbm, out_hbm, nd_v, src_v, dst_v,
          idx_v, ones_v, zv, acc):
        cid = lax.axis_index("c")
        sid = lax.axis_index("s")
        ebase = (cid * NS + sid) * EPT
        pltpu.sync_copy(nd_hbm, nd_v)
        pltpu.sync_copy(src_hbm.at[pl.ds(ebase, EPT)], src_v)
        pltpu.sync_copy(dst_hbm.at[pl.ds(ebase, EPT)], dst_v)

        def zbody(i, _):
            zv[pl.ds(i * 16, 16)] = jnp.zeros((16,), jnp.float32)
            return ()

        lax.fori_loop(0, EPT // 16, zbody, (), unroll=False)
        for q in range(CPT // EPT):
            pltpu.sync_copy(zv, acc.at[pl.ds(sid * CPT + q * EPT, EPT)])
        for j in range(CHUNK // 16):
            ones_v[pl.ds(j * 16, 16)] = jnp.ones((16,), jnp.float32)
        plsc.subcore_barrier()

        def body(c, _):
            for j in range(CHUNK // 16):
                o = c * CHUNK + j * 16
                s16 = src_v[pl.ds(o, 16)]
                d16 = dst_v[pl.ds(o, 16)]
                deg16 = plsc.load_gather(nd_v, [s16])
                idx_v[pl.ds(j * 16, 16)] = d16 * CP + deg16
            pltpu.sync_copy(ones_v, acc.at[idx_v], add=True)
            return ()

        lax.fori_loop(0, NCHUNK, body, (), unroll=False)
        plsc.subcore_barrier()
        for q in range(CPT // EPT):
            pltpu.sync_copy(acc.at[pl.ds(sid * CPT + q * EPT, EPT)], zv)
            pltpu.sync_copy(zv, out_hbm.at[
                pl.ds(cid * CFL + sid * CPT + q * EPT, EPT)])

    return k(node_deg, src, dst)


NBUF = 4


def _segsum_sc(x, edge_index):
    @functools.partial(
        pl.kernel,
        out_type=jax.ShapeDtypeStruct((NC, NP, D), jnp.float32),
        mesh=_mesh,
        compiler_params=pltpu.CompilerParams(needs_layout_passes=False),
        scratch_types=(
            [pltpu.VMEM((2, CHUNK), jnp.int32) for _ in range(NBUF)]
            + [pltpu.VMEM((CHUNK, D), jnp.float32) for _ in range(NBUF)]
            + [pltpu.VMEM_SHARED((NP, D), jnp.float32)]
            + [pltpu.SemaphoreType.DMA for _ in range(2 * NBUF)]
        ),
    )
    def k(x_hbm, ei_hbm, out_hbm, e0, e1, e2, e3, r0, r1, r2, r3, acc,
          es0, es1, es2, es3, rs0, rs1, rs2, rs3):
        eb = [e0, e1, e2, e3]
        rb = [r0, r1, r2, r3]
        es = [es0, es1, es2, es3]
        rs = [rs0, rs1, rs2, rs3]
        cid = lax.axis_index("c")
        sid = lax.axis_index("s")
        ebase = (cid * NS + sid) * EPT
        rbase = sid * RPT

        def zbody(i, _):
            for j in range(D // 16):
                r0[i, pl.ds(j * 16, 16)] = jnp.zeros((16,), jnp.float32)
            return ()

        lax.fori_loop(0, CHUNK, zbody, (), unroll=False)
        for q in range(RPT // CHUNK):
            pltpu.sync_copy(r0, acc.at[pl.ds(rbase + q * CHUNK, CHUNK)])
        plsc.subcore_barrier()

        def edge_start(batch, b):
            o = ebase + batch * CHUNK
            pltpu.async_copy(ei_hbm.at[pl.ds(o, CHUNK)], eb[b].at[0], es[b])
            pltpu.async_copy(ei_hbm.at[pl.ds(E + o, CHUNK)], eb[b].at[1],
                             es[b])

        def edge_wait(b):
            pltpu.make_async_copy(
                ei_hbm.at[pl.ds(ebase, CHUNK)], eb[b].at[0], es[b]).wait()
            pltpu.make_async_copy(
                ei_hbm.at[pl.ds(ebase, CHUNK)], eb[b].at[1], es[b]).wait()

        def gather_start(batch, b):
            pltpu.async_copy(x_hbm.at[eb[b].at[0]], rb[b], rs[b])

        def gather_wait(b):
            pltpu.make_async_copy(x_hbm.at[eb[b].at[0]], rb[b], rs[b]).wait()

        for b in range(NBUF):
            edge_start(b, b)
        for b in range(2):
            edge_wait(b)
            gather_start(b, b)

        @pl.loop(0, NCHUNK - 1, step=NBUF)
        def body(c):
            for b in range(NBUF):
                n = c + b
                gather_wait(b)
                pltpu.sync_copy(rb[b], acc.at[eb[b].at[1]], add=True)

                @pl.when(n + NBUF < NCHUNK)
                def _():
                    edge_start(n + NBUF, b)

                bp = (b + 2) % NBUF

                @pl.when(n + 2 < NCHUNK)
                def _():
                    edge_wait(bp)
                    gather_start(n + 2, bp)

        lb = (NCHUNK - 1) % NBUF
        gather_wait(lb)
        pltpu.sync_copy(rb[lb], acc.at[eb[lb].at[1]], add=True)

        plsc.subcore_barrier()
        for q in range(RPT // CHUNK):
            pltpu.sync_copy(acc.at[pl.ds(rbase + q * CHUNK, CHUNK)], r0)
            pltpu.sync_copy(r0, out_hbm.at[cid, pl.ds(rbase + q * CHUNK,
                                                      CHUNK)])

    return k(x, edge_index)


RB = 400
NRB = N // RB


def _l1_tc(degf, cpart, embed_pad, W1, b1, s1):
    def body(degf_r, p0_r, p1_r, emb_r, w_r, b_r, s_r, y_r, ssum_r, ssq_r,
             t_r):
        i = pl.program_id(0)

        @pl.when(i == 0)
        def _():
            t_r[...] = jnp.dot(emb_r[...], w_r[...].T,
                               preferred_element_type=jnp.float32)
            ssum_r[...] = jnp.zeros((1, D), jnp.float32)
            ssq_r[...] = jnp.zeros((1, D), jnp.float32)

        io = lax.broadcasted_iota(jnp.int32, (RB, CP), 1)
        onehot = jnp.where(degf_r[...].astype(jnp.int32) == io, s_r[0, 0], 0.0)
        ceff = p0_r[0] + p1_r[0] + onehot
        y = jnp.dot(ceff, t_r[...], preferred_element_type=jnp.float32)
        y = y + b_r[...]
        y_r[...] = y
        ssum_r[...] += jnp.sum(y, axis=0, keepdims=True)
        ssq_r[...] += jnp.sum(y * y, axis=0, keepdims=True)

    return pl.pallas_call(
        body,
        grid=(NRB,),
        in_specs=[
            pl.BlockSpec((RB, 1), lambda i: (i, 0)),
            pl.BlockSpec((1, RB, CP), lambda i: (0, i, 0)),
            pl.BlockSpec((1, RB, CP), lambda i: (1, i, 0)),
            pl.BlockSpec((CP, D), lambda i: (0, 0)),
            pl.BlockSpec((D, D), lambda i: (0, 0)),
            pl.BlockSpec((1, D), lambda i: (0, 0)),
            pl.BlockSpec(memory_space=pltpu.SMEM),
        ],
        out_specs=[
            pl.BlockSpec((RB, D), lambda i: (i, 0)),
            pl.BlockSpec((1, D), lambda i: (0, 0)),
            pl.BlockSpec((1, D), lambda i: (0, 0)),
        ],
        out_shape=[
            jax.ShapeDtypeStruct((N, D), jnp.float32),
            jax.ShapeDtypeStruct((1, D), jnp.float32),
            jax.ShapeDtypeStruct((1, D), jnp.float32),
        ],
        scratch_shapes=[pltpu.VMEM((CP, D), jnp.float32)],
    )(degf, cpart, cpart, embed_pad, W1, b1, s1)


def _conv_tc(x, p, W, b, s, want_stats):
    def body(x_r, p0_r, p1_r, w_r, b_r, s_r, y_r, ssum_r, ssq_r):
        i = pl.program_id(0)

        @pl.when(i == 0)
        def _():
            ssum_r[...] = jnp.zeros((1, D), jnp.float32)
            ssq_r[...] = jnp.zeros((1, D), jnp.float32)

        h = s_r[0, 0] * x_r[...] + p0_r[0] + p1_r[0]
        y = jnp.dot(h, w_r[...].T, preferred_element_type=jnp.float32)
        y = y + b_r[...]
        y_r[...] = y
        if want_stats:
            ssum_r[...] += jnp.sum(y, axis=0, keepdims=True)
            ssq_r[...] += jnp.sum(y * y, axis=0, keepdims=True)

    return pl.pallas_call(
        body,
        grid=(NRB,),
        in_specs=[
            pl.BlockSpec((RB, D), lambda i: (i, 0)),
            pl.BlockSpec((1, RB, D), lambda i: (0, i, 0)),
            pl.BlockSpec((1, RB, D), lambda i: (1, i, 0)),
            pl.BlockSpec((D, D), lambda i: (0, 0)),
            pl.BlockSpec((1, D), lambda i: (0, 0)),
            pl.BlockSpec(memory_space=pltpu.SMEM),
        ],
        out_specs=[
            pl.BlockSpec((RB, D), lambda i: (i, 0)),
            pl.BlockSpec((1, D), lambda i: (0, 0)),
            pl.BlockSpec((1, D), lambda i: (0, 0)),
        ],
        out_shape=[
            jax.ShapeDtypeStruct((N, D), jnp.float32),
            jax.ShapeDtypeStruct((1, D), jnp.float32),
            jax.ShapeDtypeStruct((1, D), jnp.float32),
        ],
    )(x, p, p, W, b, s)


def _bn_act_tc(y, ssum, ssq, g, be):
    def body(y_r, ssum_r, ssq_r, g_r, be_r, o_r):
        m = ssum_r[...] / N
        v = ssq_r[...] / N - m * m
        inv = g_r[...] * lax.rsqrt(v + 1e-5)
        xn = (y_r[...] - m) * inv + be_r[...]
        o_r[...] = jnp.where(xn >= 0, xn, 0.01 * xn)

    return pl.pallas_call(
        body,
        grid=(NRB,),
        in_specs=[
            pl.BlockSpec((RB, D), lambda i: (i, 0)),
            pl.BlockSpec((1, D), lambda i: (0, 0)),
            pl.BlockSpec((1, D), lambda i: (0, 0)),
            pl.BlockSpec((1, D), lambda i: (0, 0)),
            pl.BlockSpec((1, D), lambda i: (0, 0)),
        ],
        out_specs=pl.BlockSpec((RB, D), lambda i: (i, 0)),
        out_shape=jax.ShapeDtypeStruct((N, D), jnp.float32),
    )(y, ssum, ssq, g, be)


def kernel(node_deg, edge_index, embed_table, W1, b1, eps1, W2, b2, eps2,
           W3, b3, eps3, g1, be1, g2, be2):
    node_deg = node_deg.astype(jnp.int32)
    ei32 = edge_index.astype(jnp.int32)
    ei_flat = ei32.reshape(2 * E)
    src = ei32[0]
    dst = ei32[1]
    embed_pad = jnp.zeros((CP, D), jnp.float32).at[:embed_table.shape[0]].set(
        embed_table)
    degf = node_deg.astype(jnp.float32).reshape(N, 1)
    b1r, b2r, b3r = b1.reshape(1, D), b2.reshape(1, D), b3.reshape(1, D)
    g1r, g2r = g1.reshape(1, D), g2.reshape(1, D)
    be1r, be2r = be1.reshape(1, D), be2.reshape(1, D)
    s1 = (1.0 + eps1).astype(jnp.float32).reshape(1, 1)
    s2 = (1.0 + eps2).astype(jnp.float32).reshape(1, 1)
    s3 = (1.0 + eps3).astype(jnp.float32).reshape(1, 1)

    cpart = _hist_sc(node_deg, src, dst).reshape(NC, N, CP)
    y1, ssum1, ssq1 = _l1_tc(degf, cpart, embed_pad, W1, b1r, s1)
    x1 = _bn_act_tc(y1, ssum1, ssq1, g1r, be1r)

    p = _segsum_sc(x1, ei_flat)
    y2, ssum2, ssq2 = _conv_tc(x1, p, W2, b2r, s2, True)
    x2 = _bn_act_tc(y2, ssum2, ssq2, g2r, be2r)

    p = _segsum_sc(x2, ei_flat)
    y3, _, _ = _conv_tc(x2, p, W3, b3r, s3, False)
    return y3


# speedup vs baseline: 11.7243x; 1.0193x over previous
import functools

import jax
import jax.numpy as jnp
from jax import lax
from jax.experimental import pallas as pl
from jax.experimental.pallas import tpu as pltpu
from jax.experimental.pallas import tpu_sc as plsc

N = 10000
E = 320000
D = 128
CP = 80

NC = 2
NS = 16
EPT = E // (NC * NS)
CHUNK = 80
NCHUNK = EPT // CHUNK
NP = 10240
RPT = NP // NS
CFL = N * CP
CPT = CFL // NS

_mesh = plsc.VectorSubcoreMesh(core_axis_name="c", subcore_axis_name="s")


def _hist_sc(node_deg, src, dst):
    @functools.partial(
        pl.kernel,
        out_type=jax.ShapeDtypeStruct((NC * CFL,), jnp.float32),
        mesh=_mesh,
        compiler_params=pltpu.CompilerParams(needs_layout_passes=False),
        scratch_types=[
            pltpu.VMEM((N,), jnp.int32),
            pltpu.VMEM((EPT,), jnp.int32),
            pltpu.VMEM((EPT,), jnp.int32),
            pltpu.VMEM((CHUNK,), jnp.int32),
            pltpu.VMEM((CHUNK,), jnp.float32),
            pltpu.VMEM((EPT,), jnp.float32),
            pltpu.VMEM_SHARED((CFL,), jnp.float32),
        ],
    )
    def k(nd_hbm, src_hbm, dst_hbm, out_hbm, nd_v, src_v, dst_v,
          idx_v, ones_v, zv, acc):
        cid = lax.axis_index("c")
        sid = lax.axis_index("s")
        ebase = (cid * NS + sid) * EPT
        pltpu.sync_copy(nd_hbm, nd_v)
        pltpu.sync_copy(src_hbm.at[pl.ds(ebase, EPT)], src_v)
        pltpu.sync_copy(dst_hbm.at[pl.ds(ebase, EPT)], dst_v)

        def zbody(i, _):
            zv[pl.ds(i * 16, 16)] = jnp.zeros((16,), jnp.float32)
            return ()

        lax.fori_loop(0, EPT // 16, zbody, (), unroll=False)
        for q in range(CPT // EPT):
            pltpu.sync_copy(zv, acc.at[pl.ds(sid * CPT + q * EPT, EPT)])
        for j in range(CHUNK // 16):
            ones_v[pl.ds(j * 16, 16)] = jnp.ones((16,), jnp.float32)
        plsc.subcore_barrier()

        def body(c, _):
            for j in range(CHUNK // 16):
                o = c * CHUNK + j * 16
                s16 = src_v[pl.ds(o, 16)]
                d16 = dst_v[pl.ds(o, 16)]
                deg16 = plsc.load_gather(nd_v, [s16])
                idx_v[pl.ds(j * 16, 16)] = d16 * CP + deg16
            pltpu.sync_copy(ones_v, acc.at[idx_v], add=True)
            return ()

        lax.fori_loop(0, NCHUNK, body, (), unroll=False)
        plsc.subcore_barrier()
        for q in range(CPT // EPT):
            pltpu.sync_copy(acc.at[pl.ds(sid * CPT + q * EPT, EPT)], zv)
            pltpu.sync_copy(zv, out_hbm.at[
                pl.ds(cid * CFL + sid * CPT + q * EPT, EPT)])

    return k(node_deg, src, dst)


NRBUF = 4
NEBUF = 8


def _segsum_sc(x, edge_index):
    @functools.partial(
        pl.kernel,
        out_type=jax.ShapeDtypeStruct((NC, NP, D), jnp.float32),
        mesh=_mesh,
        compiler_params=pltpu.CompilerParams(needs_layout_passes=False),
        scratch_types=(
            [pltpu.VMEM((2, CHUNK), jnp.int32) for _ in range(NEBUF)]
            + [pltpu.VMEM((CHUNK, D), jnp.float32) for _ in range(NRBUF)]
            + [pltpu.VMEM_SHARED((NP, D), jnp.float32)]
            + [pltpu.SemaphoreType.DMA for _ in range(NEBUF + 2 * NRBUF)]
        ),
    )
    def k(x_hbm, ei_hbm, out_hbm, *refs):
        eb = refs[:NEBUF]
        rb = refs[NEBUF:NEBUF + NRBUF]
        acc = refs[NEBUF + NRBUF]
        es = refs[NEBUF + NRBUF + 1:NEBUF + NRBUF + 1 + NEBUF]
        rs = refs[NEBUF + NRBUF + 1 + NEBUF:NEBUF + NRBUF + 1 + NEBUF + NRBUF]
        ss = refs[NEBUF + NRBUF + 1 + NEBUF + NRBUF:]
        cid = lax.axis_index("c")
        sid = lax.axis_index("s")
        ebase = (cid * NS + sid) * EPT
        rbase = sid * RPT

        def zbody(i, _):
            for j in range(D // 16):
                rb[0][i, pl.ds(j * 16, 16)] = jnp.zeros((16,), jnp.float32)
            return ()

        lax.fori_loop(0, CHUNK, zbody, (), unroll=False)
        for q in range(RPT // CHUNK):
            pltpu.sync_copy(rb[0], acc.at[pl.ds(rbase + q * CHUNK, CHUNK)])
        plsc.subcore_barrier()

        def edge_start(batch, b8):
            o = ebase + batch * CHUNK
            pltpu.async_copy(ei_hbm.at[pl.ds(o, CHUNK)], eb[b8].at[0],
                             es[b8])
            pltpu.async_copy(ei_hbm.at[pl.ds(E + o, CHUNK)], eb[b8].at[1],
                             es[b8])

        def edge_wait(b8):
            pltpu.make_async_copy(
                ei_hbm.at[pl.ds(ebase, CHUNK)], eb[b8].at[0], es[b8]).wait()
            pltpu.make_async_copy(
                ei_hbm.at[pl.ds(ebase, CHUNK)], eb[b8].at[1], es[b8]).wait()

        def gather_start(b8, b4):
            pltpu.async_copy(x_hbm.at[eb[b8].at[0]], rb[b4], rs[b4])

        def gather_wait(b4):
            pltpu.make_async_copy(x_hbm.at[pl.ds(0, CHUNK)], rb[b4],
                                  rs[b4]).wait()

        def scatter_start(b8, b4):
            pltpu.async_copy(rb[b4], acc.at[eb[b8].at[1]], ss[b4], add=True)

        def scatter_wait(b4):
            pltpu.make_async_copy(rb[b4], acc.at[pl.ds(0, CHUNK)],
                                  ss[b4]).wait()

        for b in range(6):
            edge_start(b, b)
        for b in range(2):
            edge_wait(b)
            gather_start(b, b)

        def step(n, b4, b8, in_main):
            gather_wait(b4)
            scatter_start(b8, b4)
            if in_main:
                @pl.when(n >= 2)
                def _():
                    scatter_wait((b4 + 2) % NRBUF)
            elif n >= 2:
                scatter_wait((b4 + 2) % NRBUF)

            @pl.when(n + 6 < NCHUNK)
            def _():
                edge_start(n + 6, (b8 + 6) % NEBUF)

            @pl.when(n + 2 < NCHUNK)
            def _():
                edge_wait((b8 + 2) % NEBUF)
                gather_start((b8 + 2) % NEBUF, (b4 + 2) % NRBUF)

        nmain = (NCHUNK // NEBUF) * NEBUF

        @pl.loop(0, nmain, step=NEBUF)
        def body(c):
            for b in range(NEBUF):
                step(c + b, b % NRBUF, b, True)

        for n in range(nmain, NCHUNK):
            step(n, n % NRBUF, n % NEBUF, False)
        scatter_wait((NCHUNK - 2) % NRBUF)
        scatter_wait((NCHUNK - 1) % NRBUF)

        plsc.subcore_barrier()
        for q in range(RPT // CHUNK):
            pltpu.sync_copy(acc.at[pl.ds(rbase + q * CHUNK, CHUNK)], rb[0])
            pltpu.sync_copy(rb[0], out_hbm.at[cid, pl.ds(rbase + q * CHUNK,
                                                         CHUNK)])

    return k(x, edge_index)


RB = 400
NRB = N // RB


def _l1_tc(degf, cpart, embed_pad, W1, b1, s1):
    def body(degf_r, p0_r, p1_r, emb_r, w_r, b_r, s_r, y_r, ssum_r, ssq_r,
             t_r):
        i = pl.program_id(0)

        @pl.when(i == 0)
        def _():
            t_r[...] = jnp.dot(emb_r[...], w_r[...].T,
                               preferred_element_type=jnp.float32)
            ssum_r[...] = jnp.zeros((1, D), jnp.float32)
            ssq_r[...] = jnp.zeros((1, D), jnp.float32)

        io = lax.broadcasted_iota(jnp.int32, (RB, CP), 1)
        onehot = jnp.where(degf_r[...].astype(jnp.int32) == io, s_r[0, 0], 0.0)
        ceff = p0_r[0] + p1_r[0] + onehot
        y = jnp.dot(ceff, t_r[...], preferred_element_type=jnp.float32)
        y = y + b_r[...]
        y_r[...] = y
        ssum_r[...] += jnp.sum(y, axis=0, keepdims=True)
        ssq_r[...] += jnp.sum(y * y, axis=0, keepdims=True)

    return pl.pallas_call(
        body,
        grid=(NRB,),
        in_specs=[
            pl.BlockSpec((RB, 1), lambda i: (i, 0)),
            pl.BlockSpec((1, RB, CP), lambda i: (0, i, 0)),
            pl.BlockSpec((1, RB, CP), lambda i: (1, i, 0)),
            pl.BlockSpec((CP, D), lambda i: (0, 0)),
            pl.BlockSpec((D, D), lambda i: (0, 0)),
            pl.BlockSpec((1, D), lambda i: (0, 0)),
            pl.BlockSpec(memory_space=pltpu.SMEM),
        ],
        out_specs=[
            pl.BlockSpec((RB, D), lambda i: (i, 0)),
            pl.BlockSpec((1, D), lambda i: (0, 0)),
            pl.BlockSpec((1, D), lambda i: (0, 0)),
        ],
        out_shape=[
            jax.ShapeDtypeStruct((N, D), jnp.float32),
            jax.ShapeDtypeStruct((1, D), jnp.float32),
            jax.ShapeDtypeStruct((1, D), jnp.float32),
        ],
        scratch_shapes=[pltpu.VMEM((CP, D), jnp.float32)],
    )(degf, cpart, cpart, embed_pad, W1, b1, s1)


def _conv_tc(x, p, W, b, s, want_stats):
    def body(x_r, p0_r, p1_r, w_r, b_r, s_r, y_r, ssum_r, ssq_r):
        i = pl.program_id(0)

        @pl.when(i == 0)
        def _():
            ssum_r[...] = jnp.zeros((1, D), jnp.float32)
            ssq_r[...] = jnp.zeros((1, D), jnp.float32)

        h = s_r[0, 0] * x_r[...] + p0_r[0] + p1_r[0]
        y = jnp.dot(h, w_r[...].T, preferred_element_type=jnp.float32)
        y = y + b_r[...]
        y_r[...] = y
        if want_stats:
            ssum_r[...] += jnp.sum(y, axis=0, keepdims=True)
            ssq_r[...] += jnp.sum(y * y, axis=0, keepdims=True)

    return pl.pallas_call(
        body,
        grid=(NRB,),
        in_specs=[
            pl.BlockSpec((RB, D), lambda i: (i, 0)),
            pl.BlockSpec((1, RB, D), lambda i: (0, i, 0)),
            pl.BlockSpec((1, RB, D), lambda i: (1, i, 0)),
            pl.BlockSpec((D, D), lambda i: (0, 0)),
            pl.BlockSpec((1, D), lambda i: (0, 0)),
            pl.BlockSpec(memory_space=pltpu.SMEM),
        ],
        out_specs=[
            pl.BlockSpec((RB, D), lambda i: (i, 0)),
            pl.BlockSpec((1, D), lambda i: (0, 0)),
            pl.BlockSpec((1, D), lambda i: (0, 0)),
        ],
        out_shape=[
            jax.ShapeDtypeStruct((N, D), jnp.float32),
            jax.ShapeDtypeStruct((1, D), jnp.float32),
            jax.ShapeDtypeStruct((1, D), jnp.float32),
        ],
    )(x, p, p, W, b, s)


def _bn_act_tc(y, ssum, ssq, g, be):
    def body(y_r, ssum_r, ssq_r, g_r, be_r, o_r):
        m = ssum_r[...] / N
        v = ssq_r[...] / N - m * m
        inv = g_r[...] * lax.rsqrt(v + 1e-5)
        xn = (y_r[...] - m) * inv + be_r[...]
        o_r[...] = jnp.where(xn >= 0, xn, 0.01 * xn)

    return pl.pallas_call(
        body,
        grid=(NRB,),
        in_specs=[
            pl.BlockSpec((RB, D), lambda i: (i, 0)),
            pl.BlockSpec((1, D), lambda i: (0, 0)),
            pl.BlockSpec((1, D), lambda i: (0, 0)),
            pl.BlockSpec((1, D), lambda i: (0, 0)),
            pl.BlockSpec((1, D), lambda i: (0, 0)),
        ],
        out_specs=pl.BlockSpec((RB, D), lambda i: (i, 0)),
        out_shape=jax.ShapeDtypeStruct((N, D), jnp.float32),
    )(y, ssum, ssq, g, be)


def kernel(node_deg, edge_index, embed_table, W1, b1, eps1, W2, b2, eps2,
           W3, b3, eps3, g1, be1, g2, be2):
    node_deg = node_deg.astype(jnp.int32)
    ei32 = edge_index.astype(jnp.int32)
    ei_flat = ei32.reshape(2 * E)
    src = ei32[0]
    dst = ei32[1]
    embed_pad = jnp.zeros((CP, D), jnp.float32).at[:embed_table.shape[0]].set(
        embed_table)
    degf = node_deg.astype(jnp.float32).reshape(N, 1)
    b1r, b2r, b3r = b1.reshape(1, D), b2.reshape(1, D), b3.reshape(1, D)
    g1r, g2r = g1.reshape(1, D), g2.reshape(1, D)
    be1r, be2r = be1.reshape(1, D), be2.reshape(1, D)
    s1 = (1.0 + eps1).astype(jnp.float32).reshape(1, 1)
    s2 = (1.0 + eps2).astype(jnp.float32).reshape(1, 1)
    s3 = (1.0 + eps3).astype(jnp.float32).reshape(1, 1)

    cpart = _hist_sc(node_deg, src, dst).reshape(NC, N, CP)
    y1, ssum1, ssq1 = _l1_tc(degf, cpart, embed_pad, W1, b1r, s1)
    x1 = _bn_act_tc(y1, ssum1, ssq1, g1r, be1r)

    p = _segsum_sc(x1, ei_flat)
    y2, ssum2, ssq2 = _conv_tc(x1, p, W2, b2r, s2, True)
    x2 = _bn_act_tc(y2, ssum2, ssq2, g2r, be2r)

    p = _segsum_sc(x2, ei_flat)
    y3, _, _ = _conv_tc(x2, p, W3, b3r, s3, False)
    return y3


# speedup vs baseline: 13.3086x; 1.1351x over previous
import functools

import jax
import jax.numpy as jnp
from jax import lax
from jax.experimental import pallas as pl
from jax.experimental.pallas import tpu as pltpu
from jax.experimental.pallas import tpu_sc as plsc

N = 10000
E = 320000
D = 128
CP = 80

NC = 2
NS = 16
EPT = E // (NC * NS)
CHUNK = 80
NCHUNK = EPT // CHUNK
NP = 10240
RPT = NP // NS
CFL = N * CP
CPT = CFL // NS

_mesh = plsc.VectorSubcoreMesh(core_axis_name="c", subcore_axis_name="s")


def _hist_sc(node_deg, src, dst):
    @functools.partial(
        pl.kernel,
        out_type=jax.ShapeDtypeStruct((NC * CFL,), jnp.float32),
        mesh=_mesh,
        compiler_params=pltpu.CompilerParams(needs_layout_passes=False),
        scratch_types=[
            pltpu.VMEM((N,), jnp.int32),
            pltpu.VMEM((EPT,), jnp.int32),
            pltpu.VMEM((EPT,), jnp.int32),
            pltpu.VMEM((CHUNK,), jnp.int32),
            pltpu.VMEM((CHUNK,), jnp.float32),
            pltpu.VMEM((EPT,), jnp.float32),
            pltpu.VMEM_SHARED((CFL,), jnp.float32),
        ],
    )
    def k(nd_hbm, src_hbm, dst_hbm, out_hbm, nd_v, src_v, dst_v,
          idx_v, ones_v, zv, acc):
        cid = lax.axis_index("c")
        sid = lax.axis_index("s")
        ebase = (cid * NS + sid) * EPT
        pltpu.sync_copy(nd_hbm, nd_v)
        pltpu.sync_copy(src_hbm.at[pl.ds(ebase, EPT)], src_v)
        pltpu.sync_copy(dst_hbm.at[pl.ds(ebase, EPT)], dst_v)

        def zbody(i, _):
            zv[pl.ds(i * 16, 16)] = jnp.zeros((16,), jnp.float32)
            return ()

        lax.fori_loop(0, EPT // 16, zbody, (), unroll=False)
        for q in range(CPT // EPT):
            pltpu.sync_copy(zv, acc.at[pl.ds(sid * CPT + q * EPT, EPT)])
        for j in range(CHUNK // 16):
            ones_v[pl.ds(j * 16, 16)] = jnp.ones((16,), jnp.float32)
        plsc.subcore_barrier()

        def body(c, _):
            for j in range(CHUNK // 16):
                o = c * CHUNK + j * 16
                s16 = src_v[pl.ds(o, 16)]
                d16 = dst_v[pl.ds(o, 16)]
                deg16 = plsc.load_gather(nd_v, [s16])
                idx_v[pl.ds(j * 16, 16)] = d16 * CP + deg16
            pltpu.sync_copy(ones_v, acc.at[idx_v], add=True)
            return ()

        lax.fori_loop(0, NCHUNK, body, (), unroll=False)
        plsc.subcore_barrier()
        for q in range(CPT // EPT):
            pltpu.sync_copy(acc.at[pl.ds(sid * CPT + q * EPT, EPT)], zv)
            pltpu.sync_copy(zv, out_hbm.at[
                pl.ds(cid * CFL + sid * CPT + q * EPT, EPT)])

    return k(node_deg, src, dst)


NRBUF = 4
NEBUF = 8


def _segsum_sc(x, edge_index):
    @functools.partial(
        pl.kernel,
        out_type=jax.ShapeDtypeStruct((NC, NP, D), jnp.float32),
        mesh=_mesh,
        compiler_params=pltpu.CompilerParams(needs_layout_passes=False),
        scratch_types=(
            [pltpu.VMEM((2, CHUNK), jnp.int32) for _ in range(NEBUF)]
            + [pltpu.VMEM((CHUNK, D), jnp.float32) for _ in range(NRBUF)]
            + [pltpu.VMEM_SHARED((NP, D), jnp.float32)]
            + [pltpu.SemaphoreType.DMA for _ in range(NEBUF + 2 * NRBUF)]
        ),
    )
    def k(x_hbm, ei_hbm, out_hbm, *refs):
        eb = refs[:NEBUF]
        rb = refs[NEBUF:NEBUF + NRBUF]
        acc = refs[NEBUF + NRBUF]
        es = refs[NEBUF + NRBUF + 1:NEBUF + NRBUF + 1 + NEBUF]
        rs = refs[NEBUF + NRBUF + 1 + NEBUF:NEBUF + NRBUF + 1 + NEBUF + NRBUF]
        ss = refs[NEBUF + NRBUF + 1 + NEBUF + NRBUF:]
        cid = lax.axis_index("c")
        sid = lax.axis_index("s")
        ebase = (cid * NS + sid) * EPT
        rbase = sid * RPT

        def zbody(i, _):
            for j in range(D // 16):
                rb[0][i, pl.ds(j * 16, 16)] = jnp.zeros((16,), jnp.float32)
            return ()

        lax.fori_loop(0, CHUNK, zbody, (), unroll=False)
        for q in range(RPT // CHUNK):
            pltpu.sync_copy(rb[0], acc.at[pl.ds(rbase + q * CHUNK, CHUNK)])
        plsc.subcore_barrier()

        def edge_start(batch, b8):
            o = ebase + batch * CHUNK
            pltpu.async_copy(ei_hbm.at[pl.ds(o, CHUNK)], eb[b8].at[0],
                             es[b8])
            pltpu.async_copy(ei_hbm.at[pl.ds(E + o, CHUNK)], eb[b8].at[1],
                             es[b8])

        def edge_wait(b8):
            pltpu.make_async_copy(
                ei_hbm.at[pl.ds(ebase, CHUNK)], eb[b8].at[0], es[b8]).wait()
            pltpu.make_async_copy(
                ei_hbm.at[pl.ds(ebase, CHUNK)], eb[b8].at[1], es[b8]).wait()

        def gather_start(b8, b4):
            pltpu.async_copy(x_hbm.at[eb[b8].at[0]], rb[b4], rs[b4])

        def gather_wait(b4):
            pltpu.make_async_copy(x_hbm.at[pl.ds(0, CHUNK)], rb[b4],
                                  rs[b4]).wait()

        def scatter_start(b8, b4):
            pltpu.async_copy(rb[b4], acc.at[eb[b8].at[1]], ss[b4], add=True)

        def scatter_wait(b4):
            pltpu.make_async_copy(rb[b4], acc.at[pl.ds(0, CHUNK)],
                                  ss[b4]).wait()

        for b in range(6):
            edge_start(b, b)
        for b in range(2):
            edge_wait(b)
            gather_start(b, b)

        def step(n, b4, b8, in_main):
            gather_wait(b4)
            scatter_start(b8, b4)
            if in_main:
                @pl.when(n >= 2)
                def _():
                    scatter_wait((b4 + 2) % NRBUF)
            elif n >= 2:
                scatter_wait((b4 + 2) % NRBUF)

            @pl.when(n + 6 < NCHUNK)
            def _():
                edge_start(n + 6, (b8 + 6) % NEBUF)

            @pl.when(n + 2 < NCHUNK)
            def _():
                edge_wait((b8 + 2) % NEBUF)
                gather_start((b8 + 2) % NEBUF, (b4 + 2) % NRBUF)

        nmain = (NCHUNK // NEBUF) * NEBUF

        @pl.loop(0, nmain, step=NEBUF)
        def body(c):
            for b in range(NEBUF):
                step(c + b, b % NRBUF, b, True)

        for n in range(nmain, NCHUNK):
            step(n, n % NRBUF, n % NEBUF, False)
        scatter_wait((NCHUNK - 2) % NRBUF)
        scatter_wait((NCHUNK - 1) % NRBUF)

        plsc.subcore_barrier()
        for q in range(RPT // CHUNK):
            pltpu.sync_copy(acc.at[pl.ds(rbase + q * CHUNK, CHUNK)], rb[0])
            pltpu.sync_copy(rb[0], out_hbm.at[cid, pl.ds(rbase + q * CHUNK,
                                                         CHUNK)])

    return k(x, edge_index)


RB = 1000
NRB = N // RB


def _layer_tc(xin, p, aux, Wt, b, s, g, be, is_l1):
    def body(x_r, p0_r, p1_r, aux_r, w_r, b_r, s_r, g_r, be_r, o_r,
             yv, ssum_r, ssq_r, t_r):
        ph = pl.program_id(0)
        i = pl.program_id(1)

        @pl.when((ph == 0) & (i == 0))
        def _():
            ssum_r[...] = jnp.zeros((1, D), jnp.float32)
            ssq_r[...] = jnp.zeros((1, D), jnp.float32)
            if is_l1:
                t_r[...] = jnp.dot(aux_r[...], w_r[...],
                                   preferred_element_type=jnp.float32)
            else:
                t_r[...] = w_r[...]

        @pl.when(ph == 0)
        def _():
            if is_l1:
                io = lax.broadcasted_iota(jnp.int32, (RB, CP), 1)
                onehot = jnp.where(x_r[...].astype(jnp.int32) == io,
                                   s_r[0, 0], 0.0)
                h = p0_r[0] + p1_r[0] + onehot
            else:
                h = s_r[0, 0] * x_r[...] + p0_r[0] + p1_r[0]
            y = jnp.dot(h, t_r[...], preferred_element_type=jnp.float32)
            y = y + b_r[...]
            yv[pl.ds(i * RB, RB), :] = y
            ssum_r[...] += jnp.sum(y, axis=0, keepdims=True)
            ssq_r[...] += jnp.sum(y * y, axis=0, keepdims=True)

        @pl.when(ph == 1)
        def _():
            m = ssum_r[...] / N
            v = ssq_r[...] / N - m * m
            inv = g_r[...] * lax.rsqrt(v + 1e-5)
            xn = (yv[pl.ds(i * RB, RB), :] - m) * inv + be_r[...]
            o_r[...] = jnp.where(xn >= 0, xn, 0.01 * xn)

    cw = CP if is_l1 else D
    return pl.pallas_call(
        body,
        grid=(2, NRB),
        in_specs=[
            pl.BlockSpec((RB, 1 if is_l1 else D),
                         lambda ph, i: ((1 - ph) * i, 0)),
            pl.BlockSpec((1, RB, cw), lambda ph, i: (0, (1 - ph) * i, 0)),
            pl.BlockSpec((1, RB, cw), lambda ph, i: (1, (1 - ph) * i, 0)),
            pl.BlockSpec((CP, D), lambda ph, i: (0, 0)),
            pl.BlockSpec((D, D), lambda ph, i: (0, 0)),
            pl.BlockSpec((1, D), lambda ph, i: (0, 0)),
            pl.BlockSpec(memory_space=pltpu.SMEM),
            pl.BlockSpec((1, D), lambda ph, i: (0, 0)),
            pl.BlockSpec((1, D), lambda ph, i: (0, 0)),
        ],
        out_specs=pl.BlockSpec((RB, D), lambda ph, i: (ph * i, 0)),
        out_shape=jax.ShapeDtypeStruct((N, D), jnp.float32),
        scratch_shapes=[
            pltpu.VMEM((N, D), jnp.float32),
            pltpu.VMEM((1, D), jnp.float32),
            pltpu.VMEM((1, D), jnp.float32),
            pltpu.VMEM((CP if is_l1 else D, D), jnp.float32),
        ],
    )(xin, p, p, aux, Wt, b, s, g, be)


def _conv_tc(x, p, Wt, b, s):
    def body(x_r, p0_r, p1_r, w_r, b_r, s_r, y_r):
        h = s_r[0, 0] * x_r[...] + p0_r[0] + p1_r[0]
        y = jnp.dot(h, w_r[...], preferred_element_type=jnp.float32)
        y_r[...] = y + b_r[...]

    return pl.pallas_call(
        body,
        grid=(NRB,),
        in_specs=[
            pl.BlockSpec((RB, D), lambda i: (i, 0)),
            pl.BlockSpec((1, RB, D), lambda i: (0, i, 0)),
            pl.BlockSpec((1, RB, D), lambda i: (1, i, 0)),
            pl.BlockSpec((D, D), lambda i: (0, 0)),
            pl.BlockSpec((1, D), lambda i: (0, 0)),
            pl.BlockSpec(memory_space=pltpu.SMEM),
        ],
        out_specs=pl.BlockSpec((RB, D), lambda i: (i, 0)),
        out_shape=jax.ShapeDtypeStruct((N, D), jnp.float32),
    )(x, p, p, Wt, b, s)


def kernel(node_deg, edge_index, embed_table, W1, b1, eps1, W2, b2, eps2,
           W3, b3, eps3, g1, be1, g2, be2):
    node_deg = node_deg.astype(jnp.int32)
    ei32 = edge_index.astype(jnp.int32)
    ei_flat = ei32.reshape(2 * E)
    src = ei32[0]
    dst = ei32[1]
    embed_pad = jnp.zeros((CP, D), jnp.float32).at[:embed_table.shape[0]].set(
        embed_table)
    degf = node_deg.astype(jnp.float32).reshape(N, 1)
    b1r, b2r, b3r = b1.reshape(1, D), b2.reshape(1, D), b3.reshape(1, D)
    g1r, g2r = g1.reshape(1, D), g2.reshape(1, D)
    be1r, be2r = be1.reshape(1, D), be2.reshape(1, D)
    s1 = (1.0 + eps1).astype(jnp.float32).reshape(1, 1)
    s2 = (1.0 + eps2).astype(jnp.float32).reshape(1, 1)
    s3 = (1.0 + eps3).astype(jnp.float32).reshape(1, 1)
    W1t, W2t, W3t = W1.T, W2.T, W3.T

    cpart = _hist_sc(node_deg, src, dst).reshape(NC, N, CP)
    x1 = _layer_tc(degf, cpart, embed_pad, W1t, b1r, s1, g1r, be1r, True)

    p = _segsum_sc(x1, ei_flat)
    x2 = _layer_tc(x1, p, embed_pad, W2t, b2r, s2, g2r, be2r, False)

    p = _segsum_sc(x2, ei_flat)
    return _conv_tc(x2, p, W3t, b3r, s3)


# speedup vs baseline: 13.7021x; 1.0296x over previous
import functools

import jax
import jax.numpy as jnp
from jax import lax
from jax.experimental import pallas as pl
from jax.experimental.pallas import tpu as pltpu
from jax.experimental.pallas import tpu_sc as plsc

N = 10000
E = 320000
D = 128
CP = 80

NC = 2
NS = 16
EPT = E // (NC * NS)
CHUNK = 80
NCHUNK = EPT // CHUNK
NP = 10240
RPT = NP // NS
CFL = N * CP
CPT = CFL // NS

_mesh = plsc.VectorSubcoreMesh(core_axis_name="c", subcore_axis_name="s")


def _hist_sc(node_deg, src, dst):
    @functools.partial(
        pl.kernel,
        out_type=jax.ShapeDtypeStruct((NC * CFL,), jnp.float32),
        mesh=_mesh,
        compiler_params=pltpu.CompilerParams(needs_layout_passes=False),
        scratch_types=[
            pltpu.VMEM((N,), jnp.int32),
            pltpu.VMEM((EPT,), jnp.int32),
            pltpu.VMEM((EPT,), jnp.int32),
            pltpu.VMEM((CHUNK,), jnp.int32),
            pltpu.VMEM((CHUNK,), jnp.int32),
            pltpu.VMEM((CHUNK,), jnp.int32),
            pltpu.VMEM((CHUNK,), jnp.int32),
            pltpu.VMEM((CHUNK,), jnp.float32),
            pltpu.VMEM((EPT,), jnp.float32),
            pltpu.VMEM_SHARED((CFL,), jnp.float32),
            pltpu.SemaphoreType.DMA,
            pltpu.SemaphoreType.DMA,
            pltpu.SemaphoreType.DMA,
            pltpu.SemaphoreType.DMA,
        ],
    )
    def k(nd_hbm, src_hbm, dst_hbm, out_hbm, nd_v, src_v, dst_v,
          i0, i1, i2, i3, ones_v, zv, acc, ss0, ss1, ss2, ss3):
        ib = [i0, i1, i2, i3]
        ss = [ss0, ss1, ss2, ss3]
        cid = lax.axis_index("c")
        sid = lax.axis_index("s")
        ebase = (cid * NS + sid) * EPT
        pltpu.sync_copy(nd_hbm, nd_v)
        pltpu.sync_copy(src_hbm.at[pl.ds(ebase, EPT)], src_v)
        pltpu.sync_copy(dst_hbm.at[pl.ds(ebase, EPT)], dst_v)

        def zbody(i, _):
            zv[pl.ds(i * 16, 16)] = jnp.zeros((16,), jnp.float32)
            return ()

        lax.fori_loop(0, EPT // 16, zbody, (), unroll=False)
        for q in range(CPT // EPT):
            pltpu.sync_copy(zv, acc.at[pl.ds(sid * CPT + q * EPT, EPT)])
        for j in range(CHUNK // 16):
            ones_v[pl.ds(j * 16, 16)] = jnp.ones((16,), jnp.float32)
        plsc.subcore_barrier()

        def scatter_wait(b):
            pltpu.make_async_copy(ones_v, acc.at[pl.ds(0, CHUNK)],
                                  ss[b]).wait()

        def step(n, b, in_main):
            for j in range(CHUNK // 16):
                o = n * CHUNK + j * 16
                s16 = src_v[pl.ds(o, 16)]
                d16 = dst_v[pl.ds(o, 16)]
                deg16 = plsc.load_gather(nd_v, [s16])
                ib[b][pl.ds(j * 16, 16)] = d16 * CP + deg16
            pltpu.async_copy(ones_v, acc.at[ib[b]], ss[b], add=True)
            if in_main:
                @pl.when(n >= 2)
                def _():
                    scatter_wait((b + 2) % 4)
            elif n >= 2:
                scatter_wait((b + 2) % 4)

        @pl.loop(0, NCHUNK - 1, step=4)
        def body(c):
            for b in range(4):
                step(c + b, b, True)

        step(NCHUNK - 1, (NCHUNK - 1) % 4, False)
        scatter_wait((NCHUNK - 2) % 4)
        scatter_wait((NCHUNK - 1) % 4)
        plsc.subcore_barrier()
        for q in range(CPT // EPT):
            pltpu.sync_copy(acc.at[pl.ds(sid * CPT + q * EPT, EPT)], zv)
            pltpu.sync_copy(zv, out_hbm.at[
                pl.ds(cid * CFL + sid * CPT + q * EPT, EPT)])

    return k(node_deg, src, dst)


NRBUF = 4
NEBUF = 8


def _segsum_sc(x, edge_index):
    @functools.partial(
        pl.kernel,
        out_type=jax.ShapeDtypeStruct((NC, NP, D), jnp.float32),
        mesh=_mesh,
        compiler_params=pltpu.CompilerParams(needs_layout_passes=False),
        scratch_types=(
            [pltpu.VMEM((2, CHUNK), jnp.int32) for _ in range(NEBUF)]
            + [pltpu.VMEM((CHUNK, D), jnp.float32) for _ in range(NRBUF)]
            + [pltpu.VMEM_SHARED((NP, D), jnp.float32)]
            + [pltpu.SemaphoreType.DMA for _ in range(NEBUF + 2 * NRBUF)]
        ),
    )
    def k(x_hbm, ei_hbm, out_hbm, *refs):
        eb = refs[:NEBUF]
        rb = refs[NEBUF:NEBUF + NRBUF]
        acc = refs[NEBUF + NRBUF]
        es = refs[NEBUF + NRBUF + 1:NEBUF + NRBUF + 1 + NEBUF]
        rs = refs[NEBUF + NRBUF + 1 + NEBUF:NEBUF + NRBUF + 1 + NEBUF + NRBUF]
        ss = refs[NEBUF + NRBUF + 1 + NEBUF + NRBUF:]
        cid = lax.axis_index("c")
        sid = lax.axis_index("s")
        ebase = (cid * NS + sid) * EPT
        rbase = sid * RPT

        def zbody(i, _):
            for j in range(D // 16):
                rb[0][i, pl.ds(j * 16, 16)] = jnp.zeros((16,), jnp.float32)
            return ()

        lax.fori_loop(0, CHUNK, zbody, (), unroll=False)
        for q in range(RPT // CHUNK):
            pltpu.sync_copy(rb[0], acc.at[pl.ds(rbase + q * CHUNK, CHUNK)])
        plsc.subcore_barrier()

        def edge_start(batch, b8):
            o = ebase + batch * CHUNK
            pltpu.async_copy(ei_hbm.at[pl.ds(o, CHUNK)], eb[b8].at[0],
                             es[b8])
            pltpu.async_copy(ei_hbm.at[pl.ds(E + o, CHUNK)], eb[b8].at[1],
                             es[b8])

        def edge_wait(b8):
            pltpu.make_async_copy(
                ei_hbm.at[pl.ds(ebase, CHUNK)], eb[b8].at[0], es[b8]).wait()
            pltpu.make_async_copy(
                ei_hbm.at[pl.ds(ebase, CHUNK)], eb[b8].at[1], es[b8]).wait()

        def gather_start(b8, b4):
            pltpu.async_copy(x_hbm.at[eb[b8].at[0]], rb[b4], rs[b4])

        def gather_wait(b4):
            pltpu.make_async_copy(x_hbm.at[pl.ds(0, CHUNK)], rb[b4],
                                  rs[b4]).wait()

        def scatter_start(b8, b4):
            pltpu.async_copy(rb[b4], acc.at[eb[b8].at[1]], ss[b4], add=True)

        def scatter_wait(b4):
            pltpu.make_async_copy(rb[b4], acc.at[pl.ds(0, CHUNK)],
                                  ss[b4]).wait()

        for b in range(6):
            edge_start(b, b)
        for b in range(2):
            edge_wait(b)
            gather_start(b, b)

        def step(n, b4, b8, in_main):
            gather_wait(b4)
            scatter_start(b8, b4)
            if in_main:
                @pl.when(n >= 2)
                def _():
                    scatter_wait((b4 + 2) % NRBUF)
            elif n >= 2:
                scatter_wait((b4 + 2) % NRBUF)

            @pl.when(n + 6 < NCHUNK)
            def _():
                edge_start(n + 6, (b8 + 6) % NEBUF)

            @pl.when(n + 2 < NCHUNK)
            def _():
                edge_wait((b8 + 2) % NEBUF)
                gather_start((b8 + 2) % NEBUF, (b4 + 2) % NRBUF)

        nmain = (NCHUNK // NEBUF) * NEBUF

        @pl.loop(0, nmain, step=NEBUF)
        def body(c):
            for b in range(NEBUF):
                step(c + b, b % NRBUF, b, True)

        for n in range(nmain, NCHUNK):
            step(n, n % NRBUF, n % NEBUF, False)
        scatter_wait((NCHUNK - 2) % NRBUF)
        scatter_wait((NCHUNK - 1) % NRBUF)

        plsc.subcore_barrier()
        for q in range(RPT // CHUNK):
            pltpu.sync_copy(acc.at[pl.ds(rbase + q * CHUNK, CHUNK)], rb[0])
            pltpu.sync_copy(rb[0], out_hbm.at[cid, pl.ds(rbase + q * CHUNK,
                                                         CHUNK)])

    return k(x, edge_index)


RB = 1000
NRB = N // RB


def _layer_tc(xin, p, aux, Wt, b, s, g, be, is_l1):
    def body(x_r, p0_r, p1_r, aux_r, w_r, b_r, s_r, g_r, be_r, o_r,
             yv, ssum_r, ssq_r, t_r):
        ph = pl.program_id(0)
        i = pl.program_id(1)

        @pl.when((ph == 0) & (i == 0))
        def _():
            ssum_r[...] = jnp.zeros((1, D), jnp.float32)
            ssq_r[...] = jnp.zeros((1, D), jnp.float32)
            if is_l1:
                t_r[...] = jnp.dot(aux_r[...], w_r[...],
                                   preferred_element_type=jnp.float32)
            else:
                t_r[...] = w_r[...]

        @pl.when(ph == 0)
        def _():
            if is_l1:
                io = lax.broadcasted_iota(jnp.int32, (RB, CP), 1)
                onehot = jnp.where(x_r[...].astype(jnp.int32) == io,
                                   s_r[0, 0], 0.0)
                h = p0_r[0] + p1_r[0] + onehot
            else:
                h = s_r[0, 0] * x_r[...] + p0_r[0] + p1_r[0]
            y = jnp.dot(h, t_r[...], preferred_element_type=jnp.float32)
            y = y + b_r[...]
            yv[pl.ds(i * RB, RB), :] = y
            ssum_r[...] += jnp.sum(y, axis=0, keepdims=True)
            ssq_r[...] += jnp.sum(y * y, axis=0, keepdims=True)

        @pl.when(ph == 1)
        def _():
            m = ssum_r[...] / N
            v = ssq_r[...] / N - m * m
            inv = g_r[...] * lax.rsqrt(v + 1e-5)
            xn = (yv[pl.ds(i * RB, RB), :] - m) * inv + be_r[...]
            o_r[...] = jnp.where(xn >= 0, xn, 0.01 * xn)

    cw = CP if is_l1 else D
    return pl.pallas_call(
        body,
        grid=(2, NRB),
        in_specs=[
            pl.BlockSpec((RB, 1 if is_l1 else D),
                         lambda ph, i: ((1 - ph) * i, 0)),
            pl.BlockSpec((1, RB, cw), lambda ph, i: (0, (1 - ph) * i, 0)),
            pl.BlockSpec((1, RB, cw), lambda ph, i: (1, (1 - ph) * i, 0)),
            pl.BlockSpec((CP, D), lambda ph, i: (0, 0)),
            pl.BlockSpec((D, D), lambda ph, i: (0, 0)),
            pl.BlockSpec((1, D), lambda ph, i: (0, 0)),
            pl.BlockSpec(memory_space=pltpu.SMEM),
            pl.BlockSpec((1, D), lambda ph, i: (0, 0)),
            pl.BlockSpec((1, D), lambda ph, i: (0, 0)),
        ],
        out_specs=pl.BlockSpec((RB, D), lambda ph, i: (ph * i, 0)),
        out_shape=jax.ShapeDtypeStruct((N, D), jnp.float32),
        scratch_shapes=[
            pltpu.VMEM((N, D), jnp.float32),
            pltpu.VMEM((1, D), jnp.float32),
            pltpu.VMEM((1, D), jnp.float32),
            pltpu.VMEM((CP if is_l1 else D, D), jnp.float32),
        ],
    )(xin, p, p, aux, Wt, b, s, g, be)


def _conv_tc(x, p, Wt, b, s):
    def body(x_r, p0_r, p1_r, w_r, b_r, s_r, y_r):
        h = s_r[0, 0] * x_r[...] + p0_r[0] + p1_r[0]
        y = jnp.dot(h, w_r[...], preferred_element_type=jnp.float32)
        y_r[...] = y + b_r[...]

    return pl.pallas_call(
        body,
        grid=(NRB,),
        in_specs=[
            pl.BlockSpec((RB, D), lambda i: (i, 0)),
            pl.BlockSpec((1, RB, D), lambda i: (0, i, 0)),
            pl.BlockSpec((1, RB, D), lambda i: (1, i, 0)),
            pl.BlockSpec((D, D), lambda i: (0, 0)),
            pl.BlockSpec((1, D), lambda i: (0, 0)),
            pl.BlockSpec(memory_space=pltpu.SMEM),
        ],
        out_specs=pl.BlockSpec((RB, D), lambda i: (i, 0)),
        out_shape=jax.ShapeDtypeStruct((N, D), jnp.float32),
    )(x, p, p, Wt, b, s)


def kernel(node_deg, edge_index, embed_table, W1, b1, eps1, W2, b2, eps2,
           W3, b3, eps3, g1, be1, g2, be2):
    node_deg = node_deg.astype(jnp.int32)
    ei32 = edge_index.astype(jnp.int32)
    ei_flat = ei32.reshape(2 * E)
    src = ei32[0]
    dst = ei32[1]
    embed_pad = jnp.zeros((CP, D), jnp.float32).at[:embed_table.shape[0]].set(
        embed_table)
    degf = node_deg.astype(jnp.float32).reshape(N, 1)
    b1r, b2r, b3r = b1.reshape(1, D), b2.reshape(1, D), b3.reshape(1, D)
    g1r, g2r = g1.reshape(1, D), g2.reshape(1, D)
    be1r, be2r = be1.reshape(1, D), be2.reshape(1, D)
    s1 = (1.0 + eps1).astype(jnp.float32).reshape(1, 1)
    s2 = (1.0 + eps2).astype(jnp.float32).reshape(1, 1)
    s3 = (1.0 + eps3).astype(jnp.float32).reshape(1, 1)
    W1t, W2t, W3t = W1.T, W2.T, W3.T

    cpart = _hist_sc(node_deg, src, dst).reshape(NC, N, CP)
    x1 = _layer_tc(degf, cpart, embed_pad, W1t, b1r, s1, g1r, be1r, True)

    p = _segsum_sc(x1, ei_flat)
    x2 = _layer_tc(x1, p, embed_pad, W2t, b2r, s2, g2r, be2r, False)

    p = _segsum_sc(x2, ei_flat)
    return _conv_tc(x2, p, W3t, b3r, s3)


# speedup vs baseline: 14.8134x; 1.0811x over previous
import functools

import jax
import jax.numpy as jnp
from jax import lax
from jax.experimental import pallas as pl
from jax.experimental.pallas import tpu as pltpu
from jax.experimental.pallas import tpu_sc as plsc

N = 10000
E = 320000
D = 128
CP = 80

NC = 2
NS = 16
EPT = E // (NC * NS)
CHUNK = 80
NCHUNK = EPT // CHUNK
NP = 10240
RPT = NP // NS
CFL = N * CP
CPT = CFL // NS

_mesh = plsc.VectorSubcoreMesh(core_axis_name="c", subcore_axis_name="s")


def _hist_sc(node_deg, src, dst):
    @functools.partial(
        pl.kernel,
        out_type=jax.ShapeDtypeStruct((NC * CFL,), jnp.float32),
        mesh=_mesh,
        compiler_params=pltpu.CompilerParams(needs_layout_passes=False),
        scratch_types=[
            pltpu.VMEM((N,), jnp.int32),
            pltpu.VMEM((EPT,), jnp.int32),
            pltpu.VMEM((EPT,), jnp.int32),
            pltpu.VMEM((CHUNK,), jnp.int32),
            pltpu.VMEM((CHUNK,), jnp.int32),
            pltpu.VMEM((CHUNK,), jnp.int32),
            pltpu.VMEM((CHUNK,), jnp.int32),
            pltpu.VMEM((CHUNK,), jnp.float32),
            pltpu.VMEM((EPT,), jnp.float32),
            pltpu.VMEM_SHARED((CFL,), jnp.float32),
            pltpu.SemaphoreType.DMA,
            pltpu.SemaphoreType.DMA,
            pltpu.SemaphoreType.DMA,
            pltpu.SemaphoreType.DMA,
        ],
    )
    def k(nd_hbm, src_hbm, dst_hbm, out_hbm, nd_v, src_v, dst_v,
          i0, i1, i2, i3, ones_v, zv, acc, ss0, ss1, ss2, ss3):
        ib = [i0, i1, i2, i3]
        ss = [ss0, ss1, ss2, ss3]
        cid = lax.axis_index("c")
        sid = lax.axis_index("s")
        ebase = (cid * NS + sid) * EPT
        pltpu.sync_copy(nd_hbm, nd_v)
        pltpu.sync_copy(src_hbm.at[pl.ds(ebase, EPT)], src_v)
        pltpu.sync_copy(dst_hbm.at[pl.ds(ebase, EPT)], dst_v)

        def zbody(i, _):
            zv[pl.ds(i * 16, 16)] = jnp.zeros((16,), jnp.float32)
            return ()

        lax.fori_loop(0, EPT // 16, zbody, (), unroll=False)
        for q in range(CPT // EPT):
            pltpu.sync_copy(zv, acc.at[pl.ds(sid * CPT + q * EPT, EPT)])
        for j in range(CHUNK // 16):
            ones_v[pl.ds(j * 16, 16)] = jnp.ones((16,), jnp.float32)
        plsc.subcore_barrier()

        def scatter_wait(b):
            pltpu.make_async_copy(ones_v, acc.at[pl.ds(0, CHUNK)],
                                  ss[b]).wait()

        def step(n, b, in_main):
            for j in range(CHUNK // 16):
                o = n * CHUNK + j * 16
                s16 = src_v[pl.ds(o, 16)]
                d16 = dst_v[pl.ds(o, 16)]
                deg16 = plsc.load_gather(nd_v, [s16])
                ib[b][pl.ds(j * 16, 16)] = d16 * CP + deg16
            pltpu.async_copy(ones_v, acc.at[ib[b]], ss[b], add=True)
            if in_main:
                @pl.when(n >= 2)
                def _():
                    scatter_wait((b + 2) % 4)
            elif n >= 2:
                scatter_wait((b + 2) % 4)

        @pl.loop(0, NCHUNK - 1, step=4)
        def body(c):
            for b in range(4):
                step(c + b, b, True)

        step(NCHUNK - 1, (NCHUNK - 1) % 4, False)
        scatter_wait((NCHUNK - 2) % 4)
        scatter_wait((NCHUNK - 1) % 4)
        plsc.subcore_barrier()
        for q in range(CPT // EPT):
            pltpu.sync_copy(acc.at[pl.ds(sid * CPT + q * EPT, EPT)], zv)
            pltpu.sync_copy(zv, out_hbm.at[
                pl.ds(cid * CFL + sid * CPT + q * EPT, EPT)])

    return k(node_deg, src, dst)


NRBUF = 4
NEBUF = 8


def _segsum_sc(x, edge_index):
    @functools.partial(
        pl.kernel,
        out_type=jax.ShapeDtypeStruct((NC, NP, D), jnp.float32),
        mesh=_mesh,
        compiler_params=pltpu.CompilerParams(needs_layout_passes=False),
        scratch_types=(
            [pltpu.VMEM((2, CHUNK), jnp.int32) for _ in range(NEBUF)]
            + [pltpu.VMEM((CHUNK, D), jnp.float32) for _ in range(NRBUF)]
            + [pltpu.VMEM_SHARED((NP, D), jnp.float32)]
            + [pltpu.SemaphoreType.DMA for _ in range(NEBUF + 2 * NRBUF)]
        ),
    )
    def k(x_hbm, ei_hbm, out_hbm, *refs):
        eb = refs[:NEBUF]
        rb = refs[NEBUF:NEBUF + NRBUF]
        acc = refs[NEBUF + NRBUF]
        es = refs[NEBUF + NRBUF + 1:NEBUF + NRBUF + 1 + NEBUF]
        rs = refs[NEBUF + NRBUF + 1 + NEBUF:NEBUF + NRBUF + 1 + NEBUF + NRBUF]
        ss = refs[NEBUF + NRBUF + 1 + NEBUF + NRBUF:]
        cid = lax.axis_index("c")
        sid = lax.axis_index("s")
        ebase = (cid * NS + sid) * EPT
        rbase = sid * RPT

        def zbody(i, _):
            for j in range(D // 16):
                rb[0][i, pl.ds(j * 16, 16)] = jnp.zeros((16,), jnp.float32)
            return ()

        lax.fori_loop(0, CHUNK, zbody, (), unroll=False)
        for q in range(RPT // CHUNK):
            pltpu.sync_copy(rb[0], acc.at[pl.ds(rbase + q * CHUNK, CHUNK)])
        plsc.subcore_barrier()

        def edge_start(batch, b8):
            o = ebase + batch * CHUNK
            pltpu.async_copy(ei_hbm.at[pl.ds(o, CHUNK)], eb[b8].at[0],
                             es[b8])
            pltpu.async_copy(ei_hbm.at[pl.ds(E + o, CHUNK)], eb[b8].at[1],
                             es[b8])

        def edge_wait(b8):
            pltpu.make_async_copy(
                ei_hbm.at[pl.ds(ebase, CHUNK)], eb[b8].at[0], es[b8]).wait()
            pltpu.make_async_copy(
                ei_hbm.at[pl.ds(ebase, CHUNK)], eb[b8].at[1], es[b8]).wait()

        def gather_start(b8, b4):
            pltpu.async_copy(x_hbm.at[eb[b8].at[0]], rb[b4], rs[b4])

        def gather_wait(b4):
            pltpu.make_async_copy(x_hbm.at[pl.ds(0, CHUNK)], rb[b4],
                                  rs[b4]).wait()

        def scatter_start(b8, b4):
            pltpu.async_copy(rb[b4], acc.at[eb[b8].at[1]], ss[b4], add=True)

        def scatter_wait(b4):
            pltpu.make_async_copy(rb[b4], acc.at[pl.ds(0, CHUNK)],
                                  ss[b4]).wait()

        for b in range(7):
            edge_start(b, b)
        for b in range(3):
            edge_wait(b)
            gather_start(b, b)

        def step(n, b4, b8, in_main):
            gather_wait(b4)
            scatter_start(b8, b4)
            if in_main:
                @pl.when(n >= 1)
                def _():
                    scatter_wait((b4 + 3) % NRBUF)
            elif n >= 1:
                scatter_wait((b4 + 3) % NRBUF)

            @pl.when(n + 7 < NCHUNK)
            def _():
                edge_start(n + 7, (b8 + 7) % NEBUF)

            @pl.when(n + 3 < NCHUNK)
            def _():
                edge_wait((b8 + 3) % NEBUF)
                gather_start((b8 + 3) % NEBUF, (b4 + 3) % NRBUF)

        nmain = (NCHUNK // NEBUF) * NEBUF

        @pl.loop(0, nmain, step=NEBUF)
        def body(c):
            for b in range(NEBUF):
                step(c + b, b % NRBUF, b, True)

        for n in range(nmain, NCHUNK):
            step(n, n % NRBUF, n % NEBUF, False)
        scatter_wait((NCHUNK - 1) % NRBUF)

        plsc.subcore_barrier()
        for q in range(RPT // CHUNK):
            pltpu.sync_copy(acc.at[pl.ds(rbase + q * CHUNK, CHUNK)], rb[0])
            pltpu.sync_copy(rb[0], out_hbm.at[cid, pl.ds(rbase + q * CHUNK,
                                                         CHUNK)])

    return k(x, edge_index)


RB = 1000
NRB = N // RB


def _layer_tc(xin, p, aux, Wt, b, s, g, be, is_l1):
    def body(x_r, p0_r, p1_r, aux_r, w_r, b_r, s_r, g_r, be_r, o_r,
             yv, ssum_r, ssq_r, t_r):
        ph = pl.program_id(0)
        i = pl.program_id(1)

        @pl.when((ph == 0) & (i == 0))
        def _():
            ssum_r[...] = jnp.zeros((1, D), jnp.float32)
            ssq_r[...] = jnp.zeros((1, D), jnp.float32)
            if is_l1:
                t_r[...] = jnp.dot(aux_r[...], w_r[...],
                                   preferred_element_type=jnp.float32)
            else:
                t_r[...] = w_r[...]

        @pl.when(ph == 0)
        def _():
            if is_l1:
                io = lax.broadcasted_iota(jnp.int32, (RB, CP), 1)
                onehot = jnp.where(x_r[...].astype(jnp.int32) == io,
                                   s_r[0, 0], 0.0)
                h = p0_r[0] + p1_r[0] + onehot
            else:
                h = s_r[0, 0] * x_r[...] + p0_r[0] + p1_r[0]
            y = jnp.dot(h, t_r[...], preferred_element_type=jnp.float32)
            y = y + b_r[...]
            yv[pl.ds(i * RB, RB), :] = y
            ssum_r[...] += jnp.sum(y, axis=0, keepdims=True)
            ssq_r[...] += jnp.sum(y * y, axis=0, keepdims=True)

        @pl.when(ph == 1)
        def _():
            m = ssum_r[...] / N
            v = ssq_r[...] / N - m * m
            inv = g_r[...] * lax.rsqrt(v + 1e-5)
            xn = (yv[pl.ds(i * RB, RB), :] - m) * inv + be_r[...]
            o_r[...] = jnp.where(xn >= 0, xn, 0.01 * xn)

    cw = CP if is_l1 else D
    return pl.pallas_call(
        body,
        grid=(2, NRB),
        in_specs=[
            pl.BlockSpec((RB, 1 if is_l1 else D),
                         lambda ph, i: ((1 - ph) * i, 0)),
            pl.BlockSpec((1, RB, cw), lambda ph, i: (0, (1 - ph) * i, 0)),
            pl.BlockSpec((1, RB, cw), lambda ph, i: (1, (1 - ph) * i, 0)),
            pl.BlockSpec((CP, D), lambda ph, i: (0, 0)),
            pl.BlockSpec((D, D), lambda ph, i: (0, 0)),
            pl.BlockSpec((1, D), lambda ph, i: (0, 0)),
            pl.BlockSpec(memory_space=pltpu.SMEM),
            pl.BlockSpec((1, D), lambda ph, i: (0, 0)),
            pl.BlockSpec((1, D), lambda ph, i: (0, 0)),
        ],
        out_specs=pl.BlockSpec((RB, D), lambda ph, i: (ph * i, 0)),
        out_shape=jax.ShapeDtypeStruct((N, D), jnp.float32),
        scratch_shapes=[
            pltpu.VMEM((N, D), jnp.float32),
            pltpu.VMEM((1, D), jnp.float32),
            pltpu.VMEM((1, D), jnp.float32),
            pltpu.VMEM((CP if is_l1 else D, D), jnp.float32),
        ],
    )(xin, p, p, aux, Wt, b, s, g, be)


def _conv_tc(x, p, Wt, b, s):
    def body(x_r, p0_r, p1_r, w_r, b_r, s_r, y_r):
        h = s_r[0, 0] * x_r[...] + p0_r[0] + p1_r[0]
        y = jnp.dot(h, w_r[...], preferred_element_type=jnp.float32)
        y_r[...] = y + b_r[...]

    return pl.pallas_call(
        body,
        grid=(NRB,),
        in_specs=[
            pl.BlockSpec((RB, D), lambda i: (i, 0)),
            pl.BlockSpec((1, RB, D), lambda i: (0, i, 0)),
            pl.BlockSpec((1, RB, D), lambda i: (1, i, 0)),
            pl.BlockSpec((D, D), lambda i: (0, 0)),
            pl.BlockSpec((1, D), lambda i: (0, 0)),
            pl.BlockSpec(memory_space=pltpu.SMEM),
        ],
        out_specs=pl.BlockSpec((RB, D), lambda i: (i, 0)),
        out_shape=jax.ShapeDtypeStruct((N, D), jnp.float32),
    )(x, p, p, Wt, b, s)


def kernel(node_deg, edge_index, embed_table, W1, b1, eps1, W2, b2, eps2,
           W3, b3, eps3, g1, be1, g2, be2):
    node_deg = node_deg.astype(jnp.int32)
    ei32 = edge_index.astype(jnp.int32)
    ei_flat = ei32.reshape(2 * E)
    src = ei32[0]
    dst = ei32[1]
    embed_pad = jnp.zeros((CP, D), jnp.float32).at[:embed_table.shape[0]].set(
        embed_table)
    degf = node_deg.astype(jnp.float32).reshape(N, 1)
    b1r, b2r, b3r = b1.reshape(1, D), b2.reshape(1, D), b3.reshape(1, D)
    g1r, g2r = g1.reshape(1, D), g2.reshape(1, D)
    be1r, be2r = be1.reshape(1, D), be2.reshape(1, D)
    s1 = (1.0 + eps1).astype(jnp.float32).reshape(1, 1)
    s2 = (1.0 + eps2).astype(jnp.float32).reshape(1, 1)
    s3 = (1.0 + eps3).astype(jnp.float32).reshape(1, 1)
    W1t, W2t, W3t = W1.T, W2.T, W3.T

    cpart = _hist_sc(node_deg, src, dst).reshape(NC, N, CP)
    x1 = _layer_tc(degf, cpart, embed_pad, W1t, b1r, s1, g1r, be1r, True)

    p = _segsum_sc(x1, ei_flat)
    x2 = _layer_tc(x1, p, embed_pad, W2t, b2r, s2, g2r, be2r, False)

    p = _segsum_sc(x2, ei_flat)
    return _conv_tc(x2, p, W3t, b3r, s3)
